# baseline (speedup 1.0000x reference)
import jax
import jax.numpy as jnp
from jax.experimental import pallas as pl
from jax.experimental.pallas import tpu as pltpu

T = 1024
I = 100
U = 100000
P = 64

BU = 25088
NU = (U + BU - 1) // BU
UP = NU * BU

BT = 128
NT = T // BT

BT2 = 256


def _matmul_kernel(oh_ref, coef_ref, cu_ref):
    j = pl.program_id(0)
    i = pl.program_id(1)

    prod = jnp.dot(oh_ref[...], coef_ref[...], preferred_element_type=jnp.float32)
    sl = pl.ds(i * BT, BT)

    @pl.when(j == 0)
    def _init():
        cu_ref[sl, :] = prod

    @pl.when(j != 0)
    def _acc():
        cu_ref[sl, :] += prod


def _epilogue_kernel(x_ref, cu_ref, out_ref):
    out_ref[...] = jnp.sum(x_ref[...] * cu_ref[...][:, None, :], axis=-1)


def _kernel_impl(x, user_onehot, coef):
    coef_p = jnp.pad(coef, ((0, UP - U), (0, 0)))

    cu = pl.pallas_call(
        _matmul_kernel,
        grid=(NU, NT),
        in_specs=[
            pl.BlockSpec((BT, None, BU), lambda j, i: (i, 0, j)),
            pl.BlockSpec((BU, P), lambda j, i: (j, 0)),
        ],
        out_specs=pl.BlockSpec((T, P), lambda j, i: (0, 0)),
        out_shape=jax.ShapeDtypeStruct((T, P), jnp.float32),
        compiler_params=pltpu.CompilerParams(
            dimension_semantics=("arbitrary", "arbitrary"),
        ),
    )(user_onehot, coef_p)

    out = pl.pallas_call(
        _epilogue_kernel,
        grid=(T // BT2,),
        in_specs=[
            pl.BlockSpec((BT2, I, P), lambda i: (i, 0, 0)),
            pl.BlockSpec((BT2, P), lambda i: (i, 0)),
        ],
        out_specs=pl.BlockSpec((BT2, I), lambda i: (i, 0)),
        out_shape=jax.ShapeDtypeStruct((T, I), jnp.float32),
        compiler_params=pltpu.CompilerParams(
            dimension_semantics=("parallel",),
        ),
    )(x, cu)

    return out


kernel = jax.jit(_kernel_impl)


# speedup vs baseline: 2.1077x; 2.1077x over previous
import jax
import jax.numpy as jnp
from jax.experimental import pallas as pl
from jax.experimental.pallas import tpu as pltpu

T = 1024
I = 100
U = 100000
P = 64

BU = 25088
NU = (U + BU - 1) // BU
UP = NU * BU

BT = 128
NT = T // BT

BT2 = 256


def _matmul_kernel(oh_ref, coef_ref, cu_ref):
    j = pl.program_id(0)
    i = pl.program_id(1)

    prod = jnp.dot(oh_ref[...], coef_ref[...], preferred_element_type=jnp.float32)
    sl = pl.ds(i * BT, BT)

    @pl.when(j == 0)
    def _init():
        cu_ref[sl, :] = prod

    @pl.when(j != 0)
    def _acc():
        cu_ref[sl, :] += prod


def _epilogue_kernel(x_ref, cu_ref, out_ref):
    out_ref[...] = jnp.sum(x_ref[...] * cu_ref[...][:, None, :], axis=-1)


def _kernel_impl(x, user_onehot, coef):
    coef_p = jnp.pad(coef, ((0, UP - U), (0, 0)))
    oh = user_onehot.reshape(T, U)

    cu = pl.pallas_call(
        _matmul_kernel,
        grid=(NU, NT),
        in_specs=[
            pl.BlockSpec((BT, BU), lambda j, i: (i, j)),
            pl.BlockSpec((BU, P), lambda j, i: (j, 0)),
        ],
        out_specs=pl.BlockSpec((T, P), lambda j, i: (0, 0)),
        out_shape=jax.ShapeDtypeStruct((T, P), jnp.float32),
        compiler_params=pltpu.CompilerParams(
            dimension_semantics=("arbitrary", "arbitrary"),
        ),
    )(oh, coef_p)

    out = pl.pallas_call(
        _epilogue_kernel,
        grid=(T // BT2,),
        in_specs=[
            pl.BlockSpec((BT2, I, P), lambda i: (i, 0, 0)),
            pl.BlockSpec((BT2, P), lambda i: (i, 0)),
        ],
        out_specs=pl.BlockSpec((BT2, I), lambda i: (i, 0)),
        out_shape=jax.ShapeDtypeStruct((T, I), jnp.float32),
        compiler_params=pltpu.CompilerParams(
            dimension_semantics=("parallel",),
        ),
    )(x, cu)

    return out


kernel = jax.jit(_kernel_impl)


# speedup vs baseline: 7.7047x; 3.6554x over previous
import jax
import jax.numpy as jnp
from jax.experimental import pallas as pl
from jax.experimental.pallas import tpu as pltpu

T = 1024
I = 100
U = 100000
P = 64

BU = 4096
NU = (U + BU - 1) // BU

BI = 100


def _matmul_kernel(cf_ref, oh_ref, cu_ref):
    j = pl.program_id(0)

    oh = oh_ref[...]
    cf = cf_ref[...]

    @pl.when(j == 0)
    def _init():
        cu_ref[...] = jnp.dot(cf, oh, preferred_element_type=jnp.float32)

    @pl.when((j != 0) & (j != NU - 1))
    def _acc():
        cu_ref[...] += jnp.dot(cf, oh, preferred_element_type=jnp.float32)

    @pl.when(j == NU - 1)
    def _acc_tail():
        limit = U - j * BU
        col = jax.lax.broadcasted_iota(jnp.int32, cf.shape, 1)
        cfm = jnp.where(col < limit, cf, 0.0)
        cu_ref[...] += jnp.dot(cfm, oh, preferred_element_type=jnp.float32)


def _epilogue_kernel(x_ref, cu_ref, out_ref):
    out_ref[...] = jnp.sum(x_ref[...] * cu_ref[...][None, :, :], axis=1)


def _kernel_impl(x, user_onehot, coef):
    ohT = user_onehot.reshape(T, U).T
    cfT = coef.T
    xT = jnp.transpose(x, (1, 2, 0))

    cuT = pl.pallas_call(
        _matmul_kernel,
        grid=(NU,),
        in_specs=[
            pl.BlockSpec((P, BU), lambda j: (0, j)),
            pl.BlockSpec((BU, T), lambda j: (j, 0)),
        ],
        out_specs=pl.BlockSpec((P, T), lambda j: (0, 0)),
        out_shape=jax.ShapeDtypeStruct((P, T), jnp.float32),
        compiler_params=pltpu.CompilerParams(
            dimension_semantics=("arbitrary",),
        ),
    )(cfT, ohT)

    outT = pl.pallas_call(
        _epilogue_kernel,
        grid=(I // BI,),
        in_specs=[
            pl.BlockSpec((BI, P, T), lambda i: (i, 0, 0)),
            pl.BlockSpec((P, T), lambda i: (0, 0)),
        ],
        out_specs=pl.BlockSpec((BI, T), lambda i: (i, 0)),
        out_shape=jax.ShapeDtypeStruct((I, T), jnp.float32),
        compiler_params=pltpu.CompilerParams(
            dimension_semantics=("parallel",),
        ),
    )(xT, cuT)

    return outT.T


kernel = jax.jit(_kernel_impl)


# speedup vs baseline: 7.7507x; 1.0060x over previous
import jax
import jax.numpy as jnp
from jax.experimental import pallas as pl
from jax.experimental.pallas import tpu as pltpu

T = 1024
I = 100
U = 100000
P = 64

BU = 2048
NU = (U + BU - 1) // BU


def _fused_kernel(cf_ref, oh_ref, x_ref, out_ref, cu_ref):
    j = pl.program_id(0)

    oh = oh_ref[...]
    cf = cf_ref[...]

    @pl.when(j == 0)
    def _init():
        cu_ref[...] = jnp.dot(cf, oh, preferred_element_type=jnp.float32)

    @pl.when((j != 0) & (j != NU - 1))
    def _acc():
        cu_ref[...] += jnp.dot(cf, oh, preferred_element_type=jnp.float32)

    @pl.when(j == NU - 1)
    def _tail():
        limit = U - j * BU
        col = jax.lax.broadcasted_iota(jnp.int32, cf.shape, 1)
        cfm = jnp.where(col < limit, cf, 0.0)
        cu = cu_ref[...] + jnp.dot(cfm, oh, preferred_element_type=jnp.float32)
        out_ref[...] = jnp.sum(x_ref[...] * cu[None, :, :], axis=1)


def _kernel_impl(x, user_onehot, coef):
    ohT = user_onehot.reshape(T, U).T
    cfT = coef.T
    xT = jnp.transpose(x, (1, 2, 0))

    outT = pl.pallas_call(
        _fused_kernel,
        grid=(NU,),
        in_specs=[
            pl.BlockSpec((P, BU), lambda j: (0, j)),
            pl.BlockSpec((BU, T), lambda j: (j, 0)),
            pl.BlockSpec((I, P, T), lambda j: (0, 0, 0)),
        ],
        out_specs=pl.BlockSpec((I, T), lambda j: (0, 0)),
        out_shape=jax.ShapeDtypeStruct((I, T), jnp.float32),
        scratch_shapes=[pltpu.VMEM((P, T), jnp.float32)],
        compiler_params=pltpu.CompilerParams(
            dimension_semantics=("arbitrary",),
        ),
    )(cfT, ohT, xT)

    return outT.T


kernel = jax.jit(_kernel_impl)
